# baseline (speedup 1.0000x reference)
import functools

import jax
import jax.numpy as jnp
from jax import lax
from jax.experimental import pallas as pl
from jax.experimental.pallas import tpu as pltpu
from jax.experimental.pallas import tpu_sc as plsc

D = 128
NC = 2
NS = 16
NW = NC * NS
B_TOTAL = 4096 * 50
B_PER_W = B_TOTAL // NW
CHUNK = 128
NCHUNK = B_PER_W // CHUNK
NBUF = 5
NITER = NCHUNK // NBUF

_MESH = plsc.VectorSubcoreMesh(core_axis_name="c", subcore_axis_name="s")


@functools.partial(
    pl.kernel,
    mesh=_MESH,
    out_type=jax.ShapeDtypeStruct((B_TOTAL, D), jnp.float32),
    scratch_types=[
        pltpu.VMEM((NCHUNK, CHUNK), jnp.int32),
        *[pltpu.VMEM((CHUNK, D), jnp.float32) for _ in range(NBUF)],
        *[pltpu.SemaphoreType.DMA for _ in range(2 * NBUF)],
    ],
)
def _embed_gather(idx_hbm, table_hbm, out_hbm, idx_v, *bufs_and_sems):
    rows = bufs_and_sems[:NBUF]
    gsem = bufs_and_sems[NBUF:2 * NBUF]
    ssem = bufs_and_sems[2 * NBUF:]
    wid = lax.axis_index("s") * NC + lax.axis_index("c")
    base = wid * B_PER_W
    pltpu.sync_copy(idx_hbm.at[wid], idx_v)

    for j in range(NBUF):
        pltpu.async_copy(table_hbm.at[idx_v.at[j]], rows[j], gsem[j])

    def body(r, carry):
        c0 = NBUF * r
        for j in range(NBUF):
            pltpu.make_async_copy(
                table_hbm.at[idx_v.at[c0 + j]], rows[j], gsem[j]).wait()
            pltpu.async_copy(
                rows[j], out_hbm.at[pl.ds(base + (c0 + j) * CHUNK, CHUNK)],
                ssem[j])

        @pl.when(r < NITER - 1)
        def _():
            for j in range(NBUF):
                pltpu.make_async_copy(
                    rows[j], out_hbm.at[pl.ds(base, CHUNK)], ssem[j]).wait()
                pltpu.async_copy(
                    table_hbm.at[idx_v.at[c0 + NBUF + j]], rows[j], gsem[j])

        return carry

    lax.fori_loop(0, NITER, body, 0)

    for j in range(NBUF):
        pltpu.make_async_copy(
            rows[j], out_hbm.at[pl.ds(base, CHUNK)], ssem[j]).wait()


def kernel(input, table):
    batch, hist = input.shape
    idx = input.reshape(NW, NCHUNK, CHUNK).astype(jnp.int32)
    out = _embed_gather(idx, table)
    return out.reshape(batch, hist, D)


# speedup vs baseline: 1.7469x; 1.7469x over previous
import functools

import jax
import jax.numpy as jnp
from jax import lax
from jax.experimental import pallas as pl
from jax.experimental.pallas import tpu as pltpu
from jax.experimental.pallas import tpu_sc as plsc

D = 128
NC = 2
NS = 16
NW = NC * NS
BATCH = 4096
HIST = 50
B_PER_W = BATCH // NW
NBUF = 4
NITER = B_PER_W // NBUF

_MESH = plsc.VectorSubcoreMesh(core_axis_name="c", subcore_axis_name="s")


@functools.partial(
    pl.kernel,
    mesh=_MESH,
    out_type=jax.ShapeDtypeStruct((BATCH, HIST, D), jnp.float32),
    scratch_types=[
        pltpu.VMEM((B_PER_W, HIST), jnp.int32),
        *[pltpu.VMEM((HIST, D), jnp.float32) for _ in range(NBUF)],
        *[pltpu.SemaphoreType.DMA for _ in range(2 * NBUF)],
    ],
)
def _embed_gather(idx_hbm, table_hbm, out_hbm, idx_v, *bufs_and_sems):
    rows = bufs_and_sems[:NBUF]
    gsem = bufs_and_sems[NBUF:2 * NBUF]
    ssem = bufs_and_sems[2 * NBUF:]
    wid = lax.axis_index("s") * NC + lax.axis_index("c")
    b0 = wid * B_PER_W
    pltpu.sync_copy(idx_hbm.at[pl.ds(b0, B_PER_W)], idx_v)

    for j in range(NBUF):
        pltpu.async_copy(table_hbm.at[idx_v.at[j]], rows[j], gsem[j])

    def body(r, carry):
        c0 = NBUF * r
        for j in range(NBUF):
            pltpu.make_async_copy(
                table_hbm.at[idx_v.at[c0 + j]], rows[j], gsem[j]).wait()
            pltpu.async_copy(rows[j], out_hbm.at[b0 + c0 + j], ssem[j])

        @pl.when(r < NITER - 1)
        def _():
            for j in range(NBUF):
                pltpu.make_async_copy(
                    rows[j], out_hbm.at[b0], ssem[j]).wait()
                pltpu.async_copy(
                    table_hbm.at[idx_v.at[c0 + NBUF + j]], rows[j], gsem[j])

        return carry

    lax.fori_loop(0, NITER, body, 0)

    for j in range(NBUF):
        pltpu.make_async_copy(rows[j], out_hbm.at[b0], ssem[j]).wait()


def kernel(input, table):
    return _embed_gather(input.astype(jnp.int32), table)


# speedup vs baseline: 3.1516x; 1.8041x over previous
import functools

import jax
import jax.numpy as jnp
from jax import lax
from jax.experimental import pallas as pl
from jax.experimental.pallas import tpu as pltpu
from jax.experimental.pallas import tpu_sc as plsc

D = 128
NC = 2
NS = 16
NW = NC * NS
BATCH = 4096
HIST = 50
B_PER_W = BATCH // NW
NBUF = 5
NITER = HIST // NBUF

_MESH = plsc.VectorSubcoreMesh(core_axis_name="c", subcore_axis_name="s")


@functools.partial(
    pl.kernel,
    mesh=_MESH,
    out_type=jax.ShapeDtypeStruct((HIST, BATCH, D), jnp.float32),
    scratch_types=[
        pltpu.VMEM((HIST, B_PER_W), jnp.int32),
        *[pltpu.VMEM((B_PER_W, D), jnp.float32) for _ in range(NBUF)],
        *[pltpu.SemaphoreType.DMA for _ in range(2 * NBUF)],
    ],
)
def _embed_gather(idx_hbm, table_hbm, out_hbm, idx_v, *bufs_and_sems):
    rows = bufs_and_sems[:NBUF]
    gsem = bufs_and_sems[NBUF:2 * NBUF]
    ssem = bufs_and_sems[2 * NBUF:]
    wid = lax.axis_index("s") * NC + lax.axis_index("c")
    b0 = wid * B_PER_W
    pltpu.sync_copy(idx_hbm.at[pl.ds(0, HIST), pl.ds(b0, B_PER_W)], idx_v)

    for j in range(NBUF):
        pltpu.async_copy(table_hbm.at[idx_v.at[j]], rows[j], gsem[j])

    def body(r, carry):
        h0 = NBUF * r
        for j in range(NBUF):
            pltpu.make_async_copy(
                table_hbm.at[idx_v.at[h0 + j]], rows[j], gsem[j]).wait()
            pltpu.async_copy(
                rows[j], out_hbm.at[h0 + j, pl.ds(b0, B_PER_W)], ssem[j])

        @pl.when(r < NITER - 1)
        def _():
            for j in range(NBUF):
                pltpu.make_async_copy(
                    rows[j], out_hbm.at[0, pl.ds(b0, B_PER_W)], ssem[j]).wait()
                pltpu.async_copy(
                    table_hbm.at[idx_v.at[h0 + NBUF + j]], rows[j], gsem[j])

        return carry

    lax.fori_loop(0, NITER, body, 0)

    for j in range(NBUF):
        pltpu.make_async_copy(
            rows[j], out_hbm.at[0, pl.ds(b0, B_PER_W)], ssem[j]).wait()


def kernel(input, table):
    out = _embed_gather(input.T.astype(jnp.int32), table)
    return out.transpose(1, 0, 2)


# speedup vs baseline: 3.2442x; 1.0294x over previous
import functools

import jax
import jax.numpy as jnp
from jax import lax
from jax.experimental import pallas as pl
from jax.experimental.pallas import tpu as pltpu
from jax.experimental.pallas import tpu_sc as plsc

D = 128
NC = 2
NS = 16
NW = NC * NS
BATCH = 4096
HIST = 50
B_PER_W = BATCH // NW
HALF = B_PER_W // 2
NBUF = 10
NITER = 2 * HIST // NBUF

_MESH = plsc.VectorSubcoreMesh(core_axis_name="c", subcore_axis_name="s")


@functools.partial(
    pl.kernel,
    mesh=_MESH,
    out_type=jax.ShapeDtypeStruct((HIST, BATCH, D), jnp.float32),
    scratch_types=[
        pltpu.VMEM((HIST, B_PER_W), jnp.int32),
        *[pltpu.VMEM((HALF, D), jnp.float32) for _ in range(NBUF)],
        *[pltpu.SemaphoreType.DMA for _ in range(2 * NBUF)],
    ],
)
def _embed_gather(idx_hbm, table_hbm, out_hbm, idx_v, *bufs_and_sems):
    rows = bufs_and_sems[:NBUF]
    gsem = bufs_and_sems[NBUF:2 * NBUF]
    ssem = bufs_and_sems[2 * NBUF:]
    wid = lax.axis_index("s") * NC + lax.axis_index("c")
    b0 = wid * B_PER_W
    pltpu.sync_copy(idx_hbm.at[pl.ds(0, HIST), pl.ds(b0, B_PER_W)], idx_v)

    def gather(r, j):
        h = 5 * r + j // 2
        sub = (j % 2) * HALF
        pltpu.async_copy(
            table_hbm.at[idx_v.at[h, pl.ds(sub, HALF)]], rows[j], gsem[j])

    def wait_gather(r, j):
        h = 5 * r + j // 2
        sub = (j % 2) * HALF
        pltpu.make_async_copy(
            table_hbm.at[idx_v.at[h, pl.ds(sub, HALF)]], rows[j],
            gsem[j]).wait()

    def store(r, j):
        h = 5 * r + j // 2
        sub = (j % 2) * HALF
        pltpu.async_copy(
            rows[j], out_hbm.at[h, pl.ds(b0 + sub, HALF)], ssem[j])

    def wait_store(j):
        pltpu.make_async_copy(
            rows[j], out_hbm.at[0, pl.ds(b0, HALF)], ssem[j]).wait()

    for j in range(NBUF):
        gather(0, j)

    def body(r, carry):
        for j in range(NBUF):
            wait_gather(r, j)
            store(r, j)

        @pl.when(r < NITER - 1)
        def _():
            for j in range(NBUF):
                wait_store(j)
                gather(r + 1, j)

        return carry

    lax.fori_loop(0, NITER, body, 0)

    for j in range(NBUF):
        wait_store(j)


def kernel(input, table):
    out = _embed_gather(input.T.astype(jnp.int32), table)
    return out.transpose(1, 0, 2)


# speedup vs baseline: 3.2668x; 1.0070x over previous
import functools

import jax
import jax.numpy as jnp
from jax import lax
from jax.experimental import pallas as pl
from jax.experimental.pallas import tpu as pltpu
from jax.experimental.pallas import tpu_sc as plsc

D = 128
NC = 2
NS = 16
NW = NC * NS
BATCH = 4096
HIST = 50
B_PER_W = BATCH // NW
HALF = B_PER_W // 2
NBUF = 10
NITER = 2 * HIST // NBUF

_MESH = plsc.VectorSubcoreMesh(core_axis_name="c", subcore_axis_name="s")


@functools.partial(
    pl.kernel,
    mesh=_MESH,
    out_type=jax.ShapeDtypeStruct((HIST, BATCH, D), jnp.float32),
    scratch_types=[
        pltpu.VMEM((HIST, B_PER_W), jnp.int32),
        *[pltpu.VMEM((HALF, D), jnp.float32) for _ in range(NBUF)],
        *[pltpu.SemaphoreType.DMA for _ in range(2 * NBUF)],
    ],
)
def _embed_gather(idx_hbm, table_hbm, out_hbm, idx_v, *bufs_and_sems):
    rows = bufs_and_sems[:NBUF]
    gsem = bufs_and_sems[NBUF:2 * NBUF]
    ssem = bufs_and_sems[2 * NBUF:]
    wid = lax.axis_index("s") * NC + lax.axis_index("c")
    b0 = wid * B_PER_W
    pltpu.sync_copy(idx_hbm.at[pl.ds(0, HIST), pl.ds(b0, B_PER_W)], idx_v)

    def gather(r, j):
        h = 5 * r + j // 2
        sub = (j % 2) * HALF
        pltpu.async_copy(
            table_hbm.at[idx_v.at[h, pl.ds(sub, HALF)]], rows[j], gsem[j])

    def wait_gather(r, j):
        h = 5 * r + j // 2
        sub = (j % 2) * HALF
        pltpu.make_async_copy(
            table_hbm.at[idx_v.at[h, pl.ds(sub, HALF)]], rows[j],
            gsem[j]).wait()

    def store(r, j):
        h = 5 * r + j // 2
        sub = (j % 2) * HALF
        pltpu.async_copy(
            rows[j], out_hbm.at[h, pl.ds(b0 + sub, HALF)], ssem[j])

    def wait_store(j):
        pltpu.make_async_copy(
            rows[j], out_hbm.at[0, pl.ds(b0, HALF)], ssem[j]).wait()

    LAG = NBUF // 2

    def gather_c(c, j):
        h = c // 2
        sub = (c % 2) * HALF
        pltpu.async_copy(
            table_hbm.at[idx_v.at[h, pl.ds(sub, HALF)]], rows[j], gsem[j])

    for j in range(LAG):
        gather(0, j)

    def body(r, carry):
        for j in range(NBUF):
            c = NBUF * r + j
            wait_gather(r, j)
            store(r, j)
            if j < LAG:
                t = j + LAG
                @pl.when(r > 0)
                def _():
                    wait_store(t)
                gather_c(c + LAG, t)
            else:
                t = j - LAG

                @pl.when(r < NITER - 1)
                def _():
                    wait_store(t)
                    gather_c(c + LAG, t)

        return carry

    lax.fori_loop(0, NITER, body, 0)

    for j in range(NBUF):
        wait_store(j)


def kernel(input, table):
    out = _embed_gather(input.T.astype(jnp.int32), table)
    return out.transpose(1, 0, 2)


# speedup vs baseline: 3.2804x; 1.0042x over previous
import functools

import jax
import jax.numpy as jnp
from jax import lax
from jax.experimental import pallas as pl
from jax.experimental.pallas import tpu as pltpu
from jax.experimental.pallas import tpu_sc as plsc

D = 128
NC = 2
NS = 16
NW = NC * NS
BATCH = 4096
HIST = 50
B_PER_W = BATCH // NW
HALF = B_PER_W // 2
NBUF = 10
NITER = 2 * HIST // NBUF

_MESH = plsc.VectorSubcoreMesh(core_axis_name="c", subcore_axis_name="s")


@functools.partial(
    pl.kernel,
    mesh=_MESH,
    out_type=jax.ShapeDtypeStruct((HIST, BATCH, D), jnp.float32),
    scratch_types=[
        pltpu.VMEM((HIST, B_PER_W), jnp.int32),
        *[pltpu.VMEM((HALF, D), jnp.float32) for _ in range(NBUF)],
        *[pltpu.SemaphoreType.DMA for _ in range(2 * NBUF)],
    ],
)
def _embed_gather(idx_hbm, table_hbm, out_hbm, idx_v, *bufs_and_sems):
    rows = bufs_and_sems[:NBUF]
    gsem = bufs_and_sems[NBUF:2 * NBUF]
    ssem = bufs_and_sems[2 * NBUF:]
    wid = lax.axis_index("s") * NC + lax.axis_index("c")
    b0 = wid * B_PER_W
    pltpu.sync_copy(idx_hbm.at[pl.ds(0, HIST), pl.ds(b0, B_PER_W)], idx_v)

    def gather(r, j):
        h = 5 * r + j // 2
        sub = (j % 2) * HALF
        pltpu.async_copy(
            table_hbm.at[idx_v.at[h, pl.ds(sub, HALF)]], rows[j], gsem[j])

    def wait_gather(r, j):
        h = 5 * r + j // 2
        sub = (j % 2) * HALF
        pltpu.make_async_copy(
            table_hbm.at[idx_v.at[h, pl.ds(sub, HALF)]], rows[j],
            gsem[j]).wait()

    def store(r, j):
        h = 5 * r + j // 2
        sub = (j % 2) * HALF
        pltpu.async_copy(
            rows[j], out_hbm.at[h, pl.ds(b0 + sub, HALF)], ssem[j])

    def wait_store(j):
        pltpu.make_async_copy(
            rows[j], out_hbm.at[0, pl.ds(b0, HALF)], ssem[j]).wait()

    LAG = 7

    def gather_c(c, j):
        h = c // 2
        sub = (c % 2) * HALF
        pltpu.async_copy(
            table_hbm.at[idx_v.at[h, pl.ds(sub, HALF)]], rows[j], gsem[j])

    for j in range(LAG):
        gather(0, j)

    def body(r, carry):
        for j in range(NBUF):
            c = NBUF * r + j
            wait_gather(r, j)
            store(r, j)
            if j < NBUF - LAG:
                t = j + LAG
                @pl.when(r > 0)
                def _():
                    wait_store(t)
                gather_c(c + LAG, t)
            else:
                t = j - (NBUF - LAG)

                @pl.when(r < NITER - 1)
                def _():
                    wait_store(t)
                    gather_c(c + LAG, t)

        return carry

    lax.fori_loop(0, NITER, body, 0)

    for j in range(NBUF):
        wait_store(j)


def kernel(input, table):
    out = _embed_gather(input.T.astype(jnp.int32), table)
    return out.transpose(1, 0, 2)


# speedup vs baseline: 3.2894x; 1.0027x over previous
import functools

import jax
import jax.numpy as jnp
from jax import lax
from jax.experimental import pallas as pl
from jax.experimental.pallas import tpu as pltpu
from jax.experimental.pallas import tpu_sc as plsc

D = 128
NC = 2
NS = 16
NW = NC * NS
BATCH = 4096
HIST = 50
B_PER_W = BATCH // NW
HALF = B_PER_W // 2
NBUF = 10
NITER = 2 * HIST // NBUF

_MESH = plsc.VectorSubcoreMesh(core_axis_name="c", subcore_axis_name="s")


@functools.partial(
    pl.kernel,
    mesh=_MESH,
    out_type=jax.ShapeDtypeStruct((HIST, BATCH, D), jnp.float32),
    scratch_types=[
        pltpu.VMEM((HIST, B_PER_W), jnp.int32),
        *[pltpu.VMEM((HALF, D), jnp.float32) for _ in range(NBUF)],
        *[pltpu.SemaphoreType.DMA for _ in range(2 * NBUF)],
    ],
)
def _embed_gather(idx_hbm, table_hbm, out_hbm, idx_v, *bufs_and_sems):
    rows = bufs_and_sems[:NBUF]
    gsem = bufs_and_sems[NBUF:2 * NBUF]
    ssem = bufs_and_sems[2 * NBUF:]
    wid = lax.axis_index("s") * NC + lax.axis_index("c")
    b0 = wid * B_PER_W
    pltpu.sync_copy(idx_hbm.at[pl.ds(0, HIST), pl.ds(b0, B_PER_W)], idx_v)

    def gather(r, j):
        h = 5 * r + j // 2
        sub = (j % 2) * HALF
        pltpu.async_copy(
            table_hbm.at[idx_v.at[h, pl.ds(sub, HALF)]], rows[j], gsem[j])

    def wait_gather(r, j):
        h = 5 * r + j // 2
        sub = (j % 2) * HALF
        pltpu.make_async_copy(
            table_hbm.at[idx_v.at[h, pl.ds(sub, HALF)]], rows[j],
            gsem[j]).wait()

    def store(r, j):
        h = 5 * r + j // 2
        sub = (j % 2) * HALF
        pltpu.async_copy(
            rows[j], out_hbm.at[h, pl.ds(b0 + sub, HALF)], ssem[j])

    def wait_store(j):
        pltpu.make_async_copy(
            rows[j], out_hbm.at[0, pl.ds(b0, HALF)], ssem[j]).wait()

    LAG = 8

    def gather_c(c, j):
        h = c // 2
        sub = (c % 2) * HALF
        pltpu.async_copy(
            table_hbm.at[idx_v.at[h, pl.ds(sub, HALF)]], rows[j], gsem[j])

    for j in range(LAG):
        gather(0, j)

    def body(r, carry):
        for j in range(NBUF):
            c = NBUF * r + j
            wait_gather(r, j)
            store(r, j)
            if j < NBUF - LAG:
                t = j + LAG
                @pl.when(r > 0)
                def _():
                    wait_store(t)
                gather_c(c + LAG, t)
            else:
                t = j - (NBUF - LAG)

                @pl.when(r < NITER - 1)
                def _():
                    wait_store(t)
                    gather_c(c + LAG, t)

        return carry

    lax.fori_loop(0, NITER, body, 0)

    for j in range(NBUF):
        wait_store(j)


def kernel(input, table):
    out = _embed_gather(input.T.astype(jnp.int32), table)
    return out.transpose(1, 0, 2)
